# baseline (device time: 27923 ns/iter reference)
import jax
import jax.numpy as jnp
from jax import lax
from jax.experimental import pallas as pl
from jax.experimental.pallas import tpu as pltpu

N_DEV = 32
NZ = 4
NP = 8


def kernel(x, w_mat):
    k_dim, k_per = x.shape
    n = w_mat.shape[1]
    m_per = k_dim // N_DEV
    half = m_per // 2

    def body(x_hbm, w_hbm, out_hbm, xv_ref, xpack_ref, g1_ref, g2_ref,
             g3_ref, wbuf_ref, out_ref, p1_send, p1_recv, p2_send, p2_recv,
             w_sems, x_sem, out_sem):
        my_i = lax.axis_index("i")
        my_z = my_i // NP
        my_p = lax.rem(my_i, NP)

        barrier_sem = pltpu.get_barrier_semaphore()
        pl.semaphore_signal(barrier_sem, 1)
        pl.semaphore_wait(barrier_sem, 1)

        for j in range(N_DEV):
            pltpu.make_async_copy(
                w_hbm.at[pl.ds(j * m_per, m_per), :],
                wbuf_ref.at[j],
                w_sems.at[j],
            ).start()

        xcopy = pltpu.make_async_copy(x_hbm, xv_ref, x_sem)
        xcopy.start()
        xcopy.wait()
        for pt in range(NP):
            for zt in range(NZ):
                j = NP * zt + pt
                xpack_ref[pt, zt] = jnp.concatenate(
                    [xv_ref[pl.ds(j * m_per, half), :],
                     xv_ref[pl.ds(j * m_per + half, half), :]],
                    axis=1,
                )

        g1_ref[my_p] = xpack_ref[my_p]

        sends1 = []
        for dp in range(1, NP):
            pt = lax.rem(my_p + dp, NP)
            peer = NP * my_z + pt
            rdma = pltpu.make_async_remote_copy(
                src_ref=xpack_ref.at[pt],
                dst_ref=g1_ref.at[my_p],
                send_sem=p1_send.at[dp],
                recv_sem=p1_recv.at[my_p],
                device_id=(peer,),
                device_id_type=pl.DeviceIdType.MESH,
            )
            rdma.start()
            sends1.append(rdma)

        for sp in range(NP):
            @pl.when(sp != my_p)
            def _():
                recv = pltpu.make_async_remote_copy(
                    src_ref=g1_ref.at[sp],
                    dst_ref=g1_ref.at[sp],
                    send_sem=p1_send.at[0],
                    recv_sem=p1_recv.at[sp],
                    device_id=(my_p,),
                    device_id_type=pl.DeviceIdType.MESH,
                )
                recv.wait_recv()

        for zt in range(NZ):
            for sp in range(NP):
                g2_ref[zt, sp] = g1_ref[sp, zt]

        sends2 = []
        for dz in range(1, NZ):
            zt = lax.rem(my_z + dz, NZ)
            peer = NP * zt + my_p
            rdma = pltpu.make_async_remote_copy(
                src_ref=g2_ref.at[zt],
                dst_ref=g3_ref.at[my_z],
                send_sem=p2_send.at[dz],
                recv_sem=p2_recv.at[my_z],
                device_id=(peer,),
                device_id_type=pl.DeviceIdType.MESH,
            )
            rdma.start()
            sends2.append(rdma)

        for sp in range(NP):
            tile_p = g1_ref[sp, my_z]
            tile = jnp.concatenate(
                [tile_p[:, :k_per], tile_p[:, k_per:]], axis=0)
            j = NP * my_z + sp
            pltpu.make_async_copy(
                w_hbm.at[pl.ds(j * m_per, m_per), :],
                wbuf_ref.at[j],
                w_sems.at[j],
            ).wait()
            part = jnp.dot(tile, wbuf_ref[j],
                           preferred_element_type=jnp.float32)
            if sp == 0:
                out_ref[:, :] = part
            else:
                out_ref[:, :] += part

        for dz in range(1, NZ):
            zp = lax.rem(my_z + dz, NZ)
            recv = pltpu.make_async_remote_copy(
                src_ref=g3_ref.at[zp],
                dst_ref=g3_ref.at[zp],
                send_sem=p2_send.at[0],
                recv_sem=p2_recv.at[zp],
                device_id=(my_z,),
                device_id_type=pl.DeviceIdType.MESH,
            )
            recv.wait_recv()
            for sp in range(NP):
                tile_p = g3_ref[zp, sp]
                tile = jnp.concatenate(
                    [tile_p[:, :k_per], tile_p[:, k_per:]], axis=0)
                j = NP * zp + sp
                pltpu.make_async_copy(
                    w_hbm.at[pl.ds(j * m_per, m_per), :],
                    wbuf_ref.at[j],
                    w_sems.at[j],
                ).wait()
                out_ref[:, :] += jnp.dot(
                    tile, wbuf_ref[j], preferred_element_type=jnp.float32)

        out_ref[:, :] = jnp.maximum(out_ref[:, :], 0.0)

        ocopy = pltpu.make_async_copy(out_ref, out_hbm, out_sem)
        ocopy.start()
        ocopy.wait()

        for rdma in sends1 + sends2:
            rdma.wait_send()

    return pl.pallas_call(
        body,
        out_shape=jax.ShapeDtypeStruct((m_per, n), jnp.float32),
        in_specs=[
            pl.BlockSpec(memory_space=pl.ANY),
            pl.BlockSpec(memory_space=pl.ANY),
        ],
        out_specs=pl.BlockSpec(memory_space=pl.ANY),
        scratch_shapes=[
            pltpu.VMEM((k_dim, k_per), jnp.float32),
            pltpu.VMEM((NP, NZ, half, 2 * k_per), jnp.float32),
            pltpu.VMEM((NP, NZ, half, 2 * k_per), jnp.float32),
            pltpu.VMEM((NZ, NP, half, 2 * k_per), jnp.float32),
            pltpu.VMEM((NZ, NP, half, 2 * k_per), jnp.float32),
            pltpu.VMEM((N_DEV, m_per, n), jnp.float32),
            pltpu.VMEM((m_per, n), jnp.float32),
            pltpu.SemaphoreType.DMA((NP,)),
            pltpu.SemaphoreType.DMA((NP,)),
            pltpu.SemaphoreType.DMA((NZ,)),
            pltpu.SemaphoreType.DMA((NZ,)),
            pltpu.SemaphoreType.DMA((N_DEV,)),
            pltpu.SemaphoreType.DMA,
            pltpu.SemaphoreType.DMA,
        ],
        compiler_params=pltpu.CompilerParams(collective_id=0),
    )(x, w_mat)
